# Initial kernel scaffold; baseline (speedup 1.0000x reference)
import jax
import jax.numpy as jnp
from jax.experimental import pallas as pl


def kernel(input, co_matrix, spatial_filter):
    raise NotImplementedError("write your pallas kernel here")


# speedup vs baseline: 82.3840x; 82.3840x over previous
import jax
import jax.numpy as jnp
from jax.experimental import pallas as pl
from jax.experimental.pallas import tpu as pltpu

_Q = 8
_K = 3
_N, _C, _H, _W = 8, 16, 256, 256
_BH = 32


def _minmax_kernel(x_ref, min_ref, max_ref):
    xb = x_ref[0].reshape(_C * _H, _W)
    mn = jnp.min(xb, axis=0, keepdims=True)
    mn = jnp.min(mn, axis=1, keepdims=True)
    mx = jnp.max(xb, axis=0, keepdims=True)
    mx = jnp.max(mx, axis=1, keepdims=True)
    min_ref[...] = jnp.broadcast_to(mn[None], (1, 1, 128))
    max_ref[...] = jnp.broadcast_to(mx[None], (1, 1, 128))


def _main_kernel(co_ref, f_ref, minv_ref, maxv_ref, x_ref, out_ref, idx_ref):
    xmin = jnp.min(minv_ref[...])
    xmax = jnp.max(maxv_ref[...])
    co = [[co_ref[8 * a + b] for b in range(8)] for a in range(8)]
    fs = [f_ref[k] for k in range(27)]

    zc = jnp.zeros((1, _BH + 2, _W), jnp.float32)
    zw = jnp.zeros((_C + 2, _BH + 2, 1), jnp.float32)
    zrow = jnp.zeros((_C, 1, _W), jnp.float32)

    nchunks = _H // _BH
    for ci in range(nchunks):
        h0 = ci * _BH
        if ci == 0:
            xc = jnp.concatenate([zrow, x_ref[0, :, : _BH + 1, :]], axis=1)
        elif ci == nchunks - 1:
            xc = jnp.concatenate([x_ref[0, :, h0 - 1 :, :], zrow], axis=1)
        else:
            xc = x_ref[0, :, h0 - 1 : h0 + _BH + 1, :]

        t = (xc - xmin) / xmax * 8.0
        idf = jnp.floor(t)
        center = idf[:, 1 : _BH + 1, :]
        idx_ref[0, :, h0 : h0 + _BH, :] = center.astype(jnp.int32)

        eq = [center == float(a) for a in range(8)]
        acc = None
        for b in range(8):
            if b == 7:
                mb = jnp.where(idf >= 7.0, xc, 0.0)
            else:
                mb = jnp.where(idf == float(b), xc, 0.0)
            mbp = jnp.concatenate([zc, mb, zc], axis=0)
            wv = [
                jnp.concatenate([zw, mbp[:, :, : _W - 1]], axis=2),
                mbp,
                jnp.concatenate([mbp[:, :, 1:], zw], axis=2),
            ]
            sb = None
            for a3 in range(3):
                for b3 in range(3):
                    for d3 in range(3):
                        tap = wv[d3][a3 : a3 + _C, b3 : b3 + _BH, :]
                        term = fs[(a3 * 3 + b3) * 3 + d3] * tap
                        sb = term if sb is None else sb + term
            gb = jnp.where(eq[7], co[7][b], 0.0)
            for a in range(6, -1, -1):
                gb = jnp.where(eq[a], co[a][b], gb)
            term = gb * sb
            acc = term if acc is None else acc + term
        out_ref[0, :, h0 : h0 + _BH, :] = acc


def kernel(input, co_matrix, spatial_filter):
    x = input
    minv, maxv = pl.pallas_call(
        _minmax_kernel,
        grid=(2, _N // 2),
        in_specs=[
            pl.BlockSpec((1, _C, _H, _W), lambda i, j: (i * (_N // 2) + j, 0, 0, 0)),
        ],
        out_specs=[
            pl.BlockSpec((1, 1, 128), lambda i, j: (i * (_N // 2) + j, 0, 0)),
            pl.BlockSpec((1, 1, 128), lambda i, j: (i * (_N // 2) + j, 0, 0)),
        ],
        out_shape=[
            jax.ShapeDtypeStruct((_N, 1, 128), jnp.float32),
            jax.ShapeDtypeStruct((_N, 1, 128), jnp.float32),
        ],
        compiler_params=pltpu.CompilerParams(
            dimension_semantics=("parallel", "arbitrary"),
        ),
    )(x)

    out, idx = pl.pallas_call(
        _main_kernel,
        grid=(_N,),
        in_specs=[
            pl.BlockSpec(memory_space=pltpu.SMEM),
            pl.BlockSpec(memory_space=pltpu.SMEM),
            pl.BlockSpec(memory_space=pltpu.VMEM),
            pl.BlockSpec(memory_space=pltpu.VMEM),
            pl.BlockSpec((1, _C, _H, _W), lambda n: (n, 0, 0, 0)),
        ],
        out_specs=[
            pl.BlockSpec((1, _C, _H, _W), lambda n: (n, 0, 0, 0)),
            pl.BlockSpec((1, _C, _H, _W), lambda n: (n, 0, 0, 0)),
        ],
        out_shape=[
            jax.ShapeDtypeStruct((_N, _C, _H, _W), jnp.float32),
            jax.ShapeDtypeStruct((_N, _C, _H, _W), jnp.int32),
        ],
        compiler_params=pltpu.CompilerParams(
            dimension_semantics=("parallel",),
            vmem_limit_bytes=56 * 1024 * 1024,
        ),
    )(co_matrix.reshape(64), spatial_filter.reshape(27), minv, maxv, x)

    return (out, co_matrix, spatial_filter, idx)


# speedup vs baseline: 107.7567x; 1.3080x over previous
import jax
import jax.numpy as jnp
from jax.experimental import pallas as pl
from jax.experimental.pallas import tpu as pltpu

_Q = 8
_K = 3
_N, _C, _H, _W = 8, 16, 256, 256
_BH = 32


def _minmax_kernel(x_ref, min_ref, max_ref):
    xb = x_ref[0].reshape(_C * _H, _W)
    mn = jnp.min(xb, axis=0, keepdims=True)
    mn = jnp.min(mn, axis=1, keepdims=True)
    mx = jnp.max(xb, axis=0, keepdims=True)
    mx = jnp.max(mx, axis=1, keepdims=True)
    min_ref[...] = jnp.broadcast_to(mn[None], (1, 1, 128))
    max_ref[...] = jnp.broadcast_to(mx[None], (1, 1, 128))


def _main_kernel(co_ref, f_ref, minv_ref, maxv_ref, x_ref, out_ref, idx_ref):
    xmin = jnp.min(minv_ref[...])
    xmax = jnp.max(maxv_ref[...])
    co = [[co_ref[8 * a + b] for b in range(8)] for a in range(8)]
    fs = [f_ref[k] for k in range(27)]

    zc = jnp.zeros((1, _BH + 2, _W), jnp.float32)
    zw = jnp.zeros((_C + 2, _BH + 2, 1), jnp.float32)
    zrow = jnp.zeros((_C, 1, _W), jnp.float32)

    nchunks = _H // _BH
    for ci in range(nchunks):
        h0 = ci * _BH
        if ci == 0:
            xc = jnp.concatenate([zrow, x_ref[0, :, : _BH + 1, :]], axis=1)
        elif ci == nchunks - 1:
            xc = jnp.concatenate([x_ref[0, :, h0 - 1 :, :], zrow], axis=1)
        else:
            xc = x_ref[0, :, h0 - 1 : h0 + _BH + 1, :]

        t = (xc - xmin) / xmax * 8.0
        idf = jnp.floor(t)
        center = idf[:, 1 : _BH + 1, :]
        idx_ref[0, :, h0 : h0 + _BH, :] = center.astype(jnp.int32)

        eq = [center == float(a) for a in range(8)]
        acc = None
        for b in range(8):
            if b == 7:
                mb = jnp.where(idf >= 7.0, xc, 0.0)
            else:
                mb = jnp.where(idf == float(b), xc, 0.0)
            mbp = jnp.concatenate([zc, mb, zc], axis=0)
            wv = [
                jnp.concatenate([zw, mbp[:, :, : _W - 1]], axis=2),
                mbp,
                jnp.concatenate([mbp[:, :, 1:], zw], axis=2),
            ]
            rows = []
            for b3 in range(3):
                ra = None
                for a3 in range(3):
                    for d3 in range(3):
                        term = fs[(a3 * 3 + b3) * 3 + d3] * wv[d3][a3 : a3 + _C, :, :]
                        ra = term if ra is None else ra + term
                rows.append(ra)
            sb = (
                rows[0][:, 0:_BH, :]
                + rows[1][:, 1 : _BH + 1, :]
                + rows[2][:, 2 : _BH + 2, :]
            )
            gb = jnp.where(eq[7], co[7][b], 0.0)
            for a in range(6, -1, -1):
                gb = jnp.where(eq[a], co[a][b], gb)
            term = gb * sb
            acc = term if acc is None else acc + term
        out_ref[0, :, h0 : h0 + _BH, :] = acc


def kernel(input, co_matrix, spatial_filter):
    x = input
    minv, maxv = pl.pallas_call(
        _minmax_kernel,
        grid=(2, _N // 2),
        in_specs=[
            pl.BlockSpec((1, _C, _H, _W), lambda i, j: (i * (_N // 2) + j, 0, 0, 0)),
        ],
        out_specs=[
            pl.BlockSpec((1, 1, 128), lambda i, j: (i * (_N // 2) + j, 0, 0)),
            pl.BlockSpec((1, 1, 128), lambda i, j: (i * (_N // 2) + j, 0, 0)),
        ],
        out_shape=[
            jax.ShapeDtypeStruct((_N, 1, 128), jnp.float32),
            jax.ShapeDtypeStruct((_N, 1, 128), jnp.float32),
        ],
        compiler_params=pltpu.CompilerParams(
            dimension_semantics=("parallel", "arbitrary"),
        ),
    )(x)

    out, idx = pl.pallas_call(
        _main_kernel,
        grid=(_N,),
        in_specs=[
            pl.BlockSpec(memory_space=pltpu.SMEM),
            pl.BlockSpec(memory_space=pltpu.SMEM),
            pl.BlockSpec(memory_space=pltpu.VMEM),
            pl.BlockSpec(memory_space=pltpu.VMEM),
            pl.BlockSpec((1, _C, _H, _W), lambda n: (n, 0, 0, 0)),
        ],
        out_specs=[
            pl.BlockSpec((1, _C, _H, _W), lambda n: (n, 0, 0, 0)),
            pl.BlockSpec((1, _C, _H, _W), lambda n: (n, 0, 0, 0)),
        ],
        out_shape=[
            jax.ShapeDtypeStruct((_N, _C, _H, _W), jnp.float32),
            jax.ShapeDtypeStruct((_N, _C, _H, _W), jnp.int32),
        ],
        compiler_params=pltpu.CompilerParams(
            dimension_semantics=("parallel",),
            vmem_limit_bytes=56 * 1024 * 1024,
        ),
    )(co_matrix.reshape(64), spatial_filter.reshape(27), minv, maxv, x)

    return (out, co_matrix, spatial_filter, idx)


# speedup vs baseline: 112.4813x; 1.0438x over previous
import jax
import jax.numpy as jnp
from jax.experimental import pallas as pl
from jax.experimental.pallas import tpu as pltpu

_Q = 8
_K = 3
_N, _C, _H, _W = 8, 16, 256, 256
_BH = 64


def _minmax_kernel(x_ref, min_ref, max_ref):
    xb = x_ref[0].reshape(_C * _H, _W)
    mn = jnp.min(xb, axis=0, keepdims=True)
    mn = jnp.min(mn, axis=1, keepdims=True)
    mx = jnp.max(xb, axis=0, keepdims=True)
    mx = jnp.max(mx, axis=1, keepdims=True)
    min_ref[...] = jnp.broadcast_to(mn[None], (1, 1, 128))
    max_ref[...] = jnp.broadcast_to(mx[None], (1, 1, 128))


def _main_kernel(co_ref, f_ref, minv_ref, maxv_ref, x_ref, out_ref, idx_ref):
    xmin = jnp.min(minv_ref[...])
    xmax = jnp.max(maxv_ref[...])
    co = [[co_ref[8 * a + b] for b in range(8)] for a in range(8)]
    fs = [f_ref[k] for k in range(27)]

    zc = jnp.zeros((1, _BH + 2, _W), jnp.float32)
    zw = jnp.zeros((_C + 2, _BH + 2, 1), jnp.float32)
    zrow = jnp.zeros((_C, 1, _W), jnp.float32)

    nchunks = _H // _BH
    for ci in range(nchunks):
        h0 = ci * _BH
        if ci == 0:
            xc = jnp.concatenate([zrow, x_ref[0, :, : _BH + 1, :]], axis=1)
        elif ci == nchunks - 1:
            xc = jnp.concatenate([x_ref[0, :, h0 - 1 :, :], zrow], axis=1)
        else:
            xc = x_ref[0, :, h0 - 1 : h0 + _BH + 1, :]

        t = (xc - xmin) / xmax * 8.0
        idf = jnp.floor(t)
        center = idf[:, 1 : _BH + 1, :]
        idx_ref[0, :, h0 : h0 + _BH, :] = center.astype(jnp.int32)

        eq = [center == float(a) for a in range(8)]
        acc = None
        for b in range(8):
            if b == 7:
                mb = jnp.where(idf >= 7.0, xc, 0.0)
            else:
                mb = jnp.where(idf == float(b), xc, 0.0)
            mbp = jnp.concatenate([zc, mb, zc], axis=0)
            wv = [
                jnp.concatenate([zw, mbp[:, :, : _W - 1]], axis=2),
                mbp,
                jnp.concatenate([mbp[:, :, 1:], zw], axis=2),
            ]
            rows = []
            for b3 in range(3):
                ra = None
                for a3 in range(3):
                    for d3 in range(3):
                        term = fs[(a3 * 3 + b3) * 3 + d3] * wv[d3][a3 : a3 + _C, :, :]
                        ra = term if ra is None else ra + term
                rows.append(ra)
            sb = (
                rows[0][:, 0:_BH, :]
                + rows[1][:, 1 : _BH + 1, :]
                + rows[2][:, 2 : _BH + 2, :]
            )
            gb = jnp.where(eq[7], co[7][b], 0.0)
            for a in range(6, -1, -1):
                gb = jnp.where(eq[a], co[a][b], gb)
            term = gb * sb
            acc = term if acc is None else acc + term
        out_ref[0, :, h0 : h0 + _BH, :] = acc


def kernel(input, co_matrix, spatial_filter):
    x = input
    minv, maxv = pl.pallas_call(
        _minmax_kernel,
        grid=(2, _N // 2),
        in_specs=[
            pl.BlockSpec((1, _C, _H, _W), lambda i, j: (i * (_N // 2) + j, 0, 0, 0)),
        ],
        out_specs=[
            pl.BlockSpec((1, 1, 128), lambda i, j: (i * (_N // 2) + j, 0, 0)),
            pl.BlockSpec((1, 1, 128), lambda i, j: (i * (_N // 2) + j, 0, 0)),
        ],
        out_shape=[
            jax.ShapeDtypeStruct((_N, 1, 128), jnp.float32),
            jax.ShapeDtypeStruct((_N, 1, 128), jnp.float32),
        ],
        compiler_params=pltpu.CompilerParams(
            dimension_semantics=("parallel", "arbitrary"),
        ),
    )(x)

    out, idx = pl.pallas_call(
        _main_kernel,
        grid=(_N,),
        in_specs=[
            pl.BlockSpec(memory_space=pltpu.SMEM),
            pl.BlockSpec(memory_space=pltpu.SMEM),
            pl.BlockSpec(memory_space=pltpu.VMEM),
            pl.BlockSpec(memory_space=pltpu.VMEM),
            pl.BlockSpec((1, _C, _H, _W), lambda n: (n, 0, 0, 0)),
        ],
        out_specs=[
            pl.BlockSpec((1, _C, _H, _W), lambda n: (n, 0, 0, 0)),
            pl.BlockSpec((1, _C, _H, _W), lambda n: (n, 0, 0, 0)),
        ],
        out_shape=[
            jax.ShapeDtypeStruct((_N, _C, _H, _W), jnp.float32),
            jax.ShapeDtypeStruct((_N, _C, _H, _W), jnp.int32),
        ],
        compiler_params=pltpu.CompilerParams(
            dimension_semantics=("parallel",),
            vmem_limit_bytes=56 * 1024 * 1024,
        ),
    )(co_matrix.reshape(64), spatial_filter.reshape(27), minv, maxv, x)

    return (out, co_matrix, spatial_filter, idx)


# speedup vs baseline: 151.4371x; 1.3463x over previous
import jax
import jax.numpy as jnp
from jax.experimental import pallas as pl
from jax.experimental.pallas import tpu as pltpu

_Q = 8
_K = 3
_N, _C, _H, _W = 8, 16, 256, 256
_BH = 64


def _minmax_kernel(x_ref, min_ref, max_ref):
    xb = x_ref[0].reshape(_C * _H, _W)
    mn = jnp.min(xb, axis=0, keepdims=True)
    mn = jnp.min(mn, axis=1, keepdims=True)
    mx = jnp.max(xb, axis=0, keepdims=True)
    mx = jnp.max(mx, axis=1, keepdims=True)
    min_ref[...] = jnp.broadcast_to(mn[None], (1, 1, 128))
    max_ref[...] = jnp.broadcast_to(mx[None], (1, 1, 128))


def _main_kernel(co_ref, f_ref, minv_ref, maxv_ref, x_ref, out_ref, idx_ref):
    xmin = jnp.min(minv_ref[...])
    xmax = jnp.max(maxv_ref[...])
    co = [[co_ref[8 * a + b] for b in range(8)] for a in range(8)]
    fsb = [f_ref[k].astype(jnp.bfloat16) for k in range(27)]

    zc = jnp.zeros((1, _BH + 2, _W), jnp.float32)
    zw = jnp.zeros((_C + 2, _BH + 2, 1), jnp.float32)
    zrow = jnp.zeros((_C, 1, _W), jnp.float32)

    nchunks = _H // _BH
    for ci in range(nchunks):
        h0 = ci * _BH
        if ci == 0:
            xc = jnp.concatenate([zrow, x_ref[0, :, : _BH + 1, :]], axis=1)
        elif ci == nchunks - 1:
            xc = jnp.concatenate([x_ref[0, :, h0 - 1 :, :], zrow], axis=1)
        else:
            xc = x_ref[0, :, h0 - 1 : h0 + _BH + 1, :]

        t = (xc - xmin) / xmax * 8.0
        idf = jnp.floor(t)
        center = idf[:, 1 : _BH + 1, :]
        idx_ref[0, :, h0 : h0 + _BH, :] = center.astype(jnp.int32)

        eq = [center == float(a) for a in range(8)]

        acc = None
        for b in range(8):
            if b == 7:
                mb = jnp.where(idf >= 7.0, xc, 0.0)
            else:
                mb = jnp.where(idf == float(b), xc, 0.0)
            mbp = jnp.concatenate([zc, mb, zc], axis=0)
            mv = [
                jnp.concatenate([zw, mbp[:, :, : _W - 1]], axis=2).astype(jnp.bfloat16),
                mbp.astype(jnp.bfloat16),
                jnp.concatenate([mbp[:, :, 1:], zw], axis=2).astype(jnp.bfloat16),
            ]
            rows = []
            for b3 in range(3):
                ra = None
                for a3 in range(3):
                    for d3 in range(3):
                        term = fsb[(a3 * 3 + b3) * 3 + d3] * mv[d3][a3 : a3 + _C, :, :]
                        ra = term if ra is None else ra + term
                rows.append(ra)
            sb = (
                rows[0][:, 0:_BH, :]
                + rows[1][:, 1 : _BH + 1, :]
                + rows[2][:, 2 : _BH + 2, :]
            ).astype(jnp.float32)
            gb = jnp.where(eq[7], co[7][b], 0.0)
            for a in range(6, -1, -1):
                gb = jnp.where(eq[a], co[a][b], gb)
            term = gb * sb
            acc = term if acc is None else acc + term
        out_ref[0, :, h0 : h0 + _BH, :] = acc


def kernel(input, co_matrix, spatial_filter):
    x = input
    minv, maxv = pl.pallas_call(
        _minmax_kernel,
        grid=(2, _N // 2),
        in_specs=[
            pl.BlockSpec((1, _C, _H, _W), lambda i, j: (i * (_N // 2) + j, 0, 0, 0)),
        ],
        out_specs=[
            pl.BlockSpec((1, 1, 128), lambda i, j: (i * (_N // 2) + j, 0, 0)),
            pl.BlockSpec((1, 1, 128), lambda i, j: (i * (_N // 2) + j, 0, 0)),
        ],
        out_shape=[
            jax.ShapeDtypeStruct((_N, 1, 128), jnp.float32),
            jax.ShapeDtypeStruct((_N, 1, 128), jnp.float32),
        ],
        compiler_params=pltpu.CompilerParams(
            dimension_semantics=("parallel", "arbitrary"),
        ),
    )(x)

    out, idx = pl.pallas_call(
        _main_kernel,
        grid=(_N,),
        in_specs=[
            pl.BlockSpec(memory_space=pltpu.SMEM),
            pl.BlockSpec(memory_space=pltpu.SMEM),
            pl.BlockSpec(memory_space=pltpu.VMEM),
            pl.BlockSpec(memory_space=pltpu.VMEM),
            pl.BlockSpec((1, _C, _H, _W), lambda n: (n, 0, 0, 0)),
        ],
        out_specs=[
            pl.BlockSpec((1, _C, _H, _W), lambda n: (n, 0, 0, 0)),
            pl.BlockSpec((1, _C, _H, _W), lambda n: (n, 0, 0, 0)),
        ],
        out_shape=[
            jax.ShapeDtypeStruct((_N, _C, _H, _W), jnp.float32),
            jax.ShapeDtypeStruct((_N, _C, _H, _W), jnp.int32),
        ],
        compiler_params=pltpu.CompilerParams(
            dimension_semantics=("parallel",),
            vmem_limit_bytes=56 * 1024 * 1024,
        ),
    )(co_matrix.reshape(64), spatial_filter.reshape(27), minv, maxv, x)

    return (out, co_matrix, spatial_filter, idx)


# speedup vs baseline: 164.7441x; 1.0879x over previous
import jax
import jax.numpy as jnp
from jax.experimental import pallas as pl
from jax.experimental.pallas import tpu as pltpu

_Q = 8
_K = 3
_N, _C, _H, _W = 8, 16, 256, 256
_BH = 64


def _minmax_kernel(x_ref, min_ref, max_ref):
    xb = x_ref[0].reshape(_C * _H, _W)
    mn = jnp.min(xb, axis=0, keepdims=True)
    mn = jnp.min(mn, axis=1, keepdims=True)
    mx = jnp.max(xb, axis=0, keepdims=True)
    mx = jnp.max(mx, axis=1, keepdims=True)
    min_ref[...] = jnp.broadcast_to(mn[None], (1, 1, 128))
    max_ref[...] = jnp.broadcast_to(mx[None], (1, 1, 128))


def _main_kernel(co_ref, f_ref, minv_ref, maxv_ref, x_ref, out_ref, idx_ref):
    xmin = jnp.min(minv_ref[...])
    xmax = jnp.max(maxv_ref[...])
    cob = [[co_ref[8 * a + b].astype(jnp.bfloat16) for b in range(8)]
           for a in range(8)]
    fsb = [f_ref[k].astype(jnp.bfloat16) for k in range(27)]

    zc = jnp.zeros((1, _BH + 2, _W), jnp.float32)
    zw = jnp.zeros((_C + 2, _BH + 2, 1), jnp.float32)
    zrow = jnp.zeros((_C, 1, _W), jnp.float32)

    nchunks = _H // _BH
    for ci in range(nchunks):
        h0 = ci * _BH
        if ci == 0:
            xc = jnp.concatenate([zrow, x_ref[0, :, : _BH + 1, :]], axis=1)
        elif ci == nchunks - 1:
            xc = jnp.concatenate([x_ref[0, :, h0 - 1 :, :], zrow], axis=1)
        else:
            xc = x_ref[0, :, h0 - 1 : h0 + _BH + 1, :]

        t = (xc - xmin) / xmax * 8.0
        idf = jnp.floor(t)
        center = idf[:, 1 : _BH + 1, :]
        idx_ref[0, :, h0 : h0 + _BH, :] = center.astype(jnp.int32)

        cb = center.astype(jnp.bfloat16)
        eq = [cb == jnp.bfloat16(a) for a in range(8)]

        acc = None
        for b in range(8):
            if b == 7:
                mb = jnp.where(idf >= 7.0, xc, 0.0)
            else:
                mb = jnp.where(idf == float(b), xc, 0.0)
            mbp = jnp.concatenate([zc, mb, zc], axis=0)
            mv = [
                jnp.concatenate([zw, mbp[:, :, : _W - 1]], axis=2).astype(jnp.bfloat16),
                mbp.astype(jnp.bfloat16),
                jnp.concatenate([mbp[:, :, 1:], zw], axis=2).astype(jnp.bfloat16),
            ]
            rows = []
            for b3 in range(3):
                ra = None
                for a3 in range(3):
                    for d3 in range(3):
                        term = fsb[(a3 * 3 + b3) * 3 + d3] * mv[d3][a3 : a3 + _C, :, :]
                        ra = term if ra is None else ra + term
                rows.append(ra)
            sb = (
                rows[0][:, 0:_BH, :]
                + rows[1][:, 1 : _BH + 1, :]
                + rows[2][:, 2 : _BH + 2, :]
            )
            gb = jnp.where(eq[7], cob[7][b], jnp.bfloat16(0))
            for a in range(6, -1, -1):
                gb = jnp.where(eq[a], cob[a][b], gb)
            term = gb * sb
            acc = term if acc is None else acc + term
        out_ref[0, :, h0 : h0 + _BH, :] = acc.astype(jnp.float32)


def kernel(input, co_matrix, spatial_filter):
    x = input
    minv, maxv = pl.pallas_call(
        _minmax_kernel,
        grid=(2, _N // 2),
        in_specs=[
            pl.BlockSpec((1, _C, _H, _W), lambda i, j: (i * (_N // 2) + j, 0, 0, 0)),
        ],
        out_specs=[
            pl.BlockSpec((1, 1, 128), lambda i, j: (i * (_N // 2) + j, 0, 0)),
            pl.BlockSpec((1, 1, 128), lambda i, j: (i * (_N // 2) + j, 0, 0)),
        ],
        out_shape=[
            jax.ShapeDtypeStruct((_N, 1, 128), jnp.float32),
            jax.ShapeDtypeStruct((_N, 1, 128), jnp.float32),
        ],
        compiler_params=pltpu.CompilerParams(
            dimension_semantics=("parallel", "arbitrary"),
        ),
    )(x)

    out, idx = pl.pallas_call(
        _main_kernel,
        grid=(_N,),
        in_specs=[
            pl.BlockSpec(memory_space=pltpu.SMEM),
            pl.BlockSpec(memory_space=pltpu.SMEM),
            pl.BlockSpec(memory_space=pltpu.VMEM),
            pl.BlockSpec(memory_space=pltpu.VMEM),
            pl.BlockSpec((1, _C, _H, _W), lambda n: (n, 0, 0, 0)),
        ],
        out_specs=[
            pl.BlockSpec((1, _C, _H, _W), lambda n: (n, 0, 0, 0)),
            pl.BlockSpec((1, _C, _H, _W), lambda n: (n, 0, 0, 0)),
        ],
        out_shape=[
            jax.ShapeDtypeStruct((_N, _C, _H, _W), jnp.float32),
            jax.ShapeDtypeStruct((_N, _C, _H, _W), jnp.int32),
        ],
        compiler_params=pltpu.CompilerParams(
            dimension_semantics=("parallel",),
            vmem_limit_bytes=56 * 1024 * 1024,
        ),
    )(co_matrix.reshape(64), spatial_filter.reshape(27), minv, maxv, x)

    return (out, co_matrix, spatial_filter, idx)


# speedup vs baseline: 173.6732x; 1.0542x over previous
import jax
import jax.numpy as jnp
from jax.experimental import pallas as pl
from jax.experimental.pallas import tpu as pltpu

_Q = 8
_K = 3
_N, _C, _H, _W = 8, 16, 256, 256
_BH = 128


def _minmax_kernel(x_ref, min_ref, max_ref):
    xb = x_ref[0].reshape(_C * _H, _W)
    mn = jnp.min(xb, axis=0, keepdims=True)
    mn = jnp.min(mn, axis=1, keepdims=True)
    mx = jnp.max(xb, axis=0, keepdims=True)
    mx = jnp.max(mx, axis=1, keepdims=True)
    min_ref[...] = jnp.broadcast_to(mn[None], (1, 1, 128))
    max_ref[...] = jnp.broadcast_to(mx[None], (1, 1, 128))


def _main_kernel(co_ref, f_ref, minv_ref, maxv_ref, x_ref, out_ref, idx_ref):
    xmin = jnp.min(minv_ref[...])
    xmax = jnp.max(maxv_ref[...])
    cob = [[co_ref[8 * a + b].astype(jnp.bfloat16) for b in range(8)]
           for a in range(8)]
    fsb = [f_ref[k].astype(jnp.bfloat16) for k in range(27)]

    zc = jnp.zeros((1, _BH + 2, _W), jnp.float32)
    zw = jnp.zeros((_C + 2, _BH + 2, 1), jnp.float32)
    zrow = jnp.zeros((_C, 1, _W), jnp.float32)

    nchunks = _H // _BH
    for ci in range(nchunks):
        h0 = ci * _BH
        if ci == 0:
            xc = jnp.concatenate([zrow, x_ref[0, :, : _BH + 1, :]], axis=1)
        elif ci == nchunks - 1:
            xc = jnp.concatenate([x_ref[0, :, h0 - 1 :, :], zrow], axis=1)
        else:
            xc = x_ref[0, :, h0 - 1 : h0 + _BH + 1, :]

        t = (xc - xmin) / xmax * 8.0
        idf = jnp.floor(t)
        center = idf[:, 1 : _BH + 1, :]
        idx_ref[0, :, h0 : h0 + _BH, :] = center.astype(jnp.int32)

        cb = center.astype(jnp.bfloat16)
        eq = [cb == jnp.bfloat16(a) for a in range(8)]

        acc = None
        for b in range(8):
            if b == 7:
                mb = jnp.where(idf >= 7.0, xc, 0.0)
            else:
                mb = jnp.where(idf == float(b), xc, 0.0)
            mbp = jnp.concatenate([zc, mb, zc], axis=0)
            mv = [
                jnp.concatenate([zw, mbp[:, :, : _W - 1]], axis=2).astype(jnp.bfloat16),
                mbp.astype(jnp.bfloat16),
                jnp.concatenate([mbp[:, :, 1:], zw], axis=2).astype(jnp.bfloat16),
            ]
            rows = []
            for b3 in range(3):
                ra = None
                for a3 in range(3):
                    for d3 in range(3):
                        term = fsb[(a3 * 3 + b3) * 3 + d3] * mv[d3][a3 : a3 + _C, :, :]
                        ra = term if ra is None else ra + term
                rows.append(ra)
            sb = (
                rows[0][:, 0:_BH, :]
                + rows[1][:, 1 : _BH + 1, :]
                + rows[2][:, 2 : _BH + 2, :]
            )
            gb = jnp.where(eq[7], cob[7][b], jnp.bfloat16(0))
            for a in range(6, -1, -1):
                gb = jnp.where(eq[a], cob[a][b], gb)
            term = gb * sb
            acc = term if acc is None else acc + term
        out_ref[0, :, h0 : h0 + _BH, :] = acc.astype(jnp.float32)


def kernel(input, co_matrix, spatial_filter):
    x = input
    minv, maxv = pl.pallas_call(
        _minmax_kernel,
        grid=(2, _N // 2),
        in_specs=[
            pl.BlockSpec((1, _C, _H, _W), lambda i, j: (i * (_N // 2) + j, 0, 0, 0)),
        ],
        out_specs=[
            pl.BlockSpec((1, 1, 128), lambda i, j: (i * (_N // 2) + j, 0, 0)),
            pl.BlockSpec((1, 1, 128), lambda i, j: (i * (_N // 2) + j, 0, 0)),
        ],
        out_shape=[
            jax.ShapeDtypeStruct((_N, 1, 128), jnp.float32),
            jax.ShapeDtypeStruct((_N, 1, 128), jnp.float32),
        ],
        compiler_params=pltpu.CompilerParams(
            dimension_semantics=("parallel", "arbitrary"),
        ),
    )(x)

    out, idx = pl.pallas_call(
        _main_kernel,
        grid=(_N,),
        in_specs=[
            pl.BlockSpec(memory_space=pltpu.SMEM),
            pl.BlockSpec(memory_space=pltpu.SMEM),
            pl.BlockSpec(memory_space=pltpu.VMEM),
            pl.BlockSpec(memory_space=pltpu.VMEM),
            pl.BlockSpec((1, _C, _H, _W), lambda n: (n, 0, 0, 0)),
        ],
        out_specs=[
            pl.BlockSpec((1, _C, _H, _W), lambda n: (n, 0, 0, 0)),
            pl.BlockSpec((1, _C, _H, _W), lambda n: (n, 0, 0, 0)),
        ],
        out_shape=[
            jax.ShapeDtypeStruct((_N, _C, _H, _W), jnp.float32),
            jax.ShapeDtypeStruct((_N, _C, _H, _W), jnp.int32),
        ],
        compiler_params=pltpu.CompilerParams(
            dimension_semantics=("parallel",),
            vmem_limit_bytes=56 * 1024 * 1024,
        ),
    )(co_matrix.reshape(64), spatial_filter.reshape(27), minv, maxv, x)

    return (out, co_matrix, spatial_filter, idx)


# speedup vs baseline: 178.7313x; 1.0291x over previous
import jax
import jax.numpy as jnp
from jax.experimental import pallas as pl
from jax.experimental.pallas import tpu as pltpu

_Q = 8
_K = 3
_N, _C, _H, _W = 8, 16, 256, 256
_BH = 256


def _minmax_kernel(x_ref, min_ref, max_ref):
    xb = x_ref[0].reshape(_C * _H, _W)
    mn = jnp.min(xb, axis=0, keepdims=True)
    mn = jnp.min(mn, axis=1, keepdims=True)
    mx = jnp.max(xb, axis=0, keepdims=True)
    mx = jnp.max(mx, axis=1, keepdims=True)
    min_ref[...] = jnp.broadcast_to(mn[None], (1, 1, 128))
    max_ref[...] = jnp.broadcast_to(mx[None], (1, 1, 128))


def _main_kernel(co_ref, f_ref, minv_ref, maxv_ref, x_ref, out_ref, idx_ref):
    xmin = jnp.min(minv_ref[...])
    xmax = jnp.max(maxv_ref[...])
    cob = [[co_ref[8 * a + b].astype(jnp.bfloat16) for b in range(8)]
           for a in range(8)]
    fsb = [f_ref[k].astype(jnp.bfloat16) for k in range(27)]

    zc = jnp.zeros((1, _BH + 2, _W), jnp.float32)
    zw = jnp.zeros((_C + 2, _BH + 2, 1), jnp.float32)
    zrow = jnp.zeros((_C, 1, _W), jnp.float32)

    nchunks = _H // _BH
    for ci in range(nchunks):
        h0 = ci * _BH
        if nchunks == 1:
            xc = jnp.concatenate([zrow, x_ref[0, :, :, :], zrow], axis=1)
        elif ci == 0:
            xc = jnp.concatenate([zrow, x_ref[0, :, : _BH + 1, :]], axis=1)
        elif ci == nchunks - 1:
            xc = jnp.concatenate([x_ref[0, :, h0 - 1 :, :], zrow], axis=1)
        else:
            xc = x_ref[0, :, h0 - 1 : h0 + _BH + 1, :]

        t = (xc - xmin) / xmax * 8.0
        idf = jnp.floor(t)
        center = idf[:, 1 : _BH + 1, :]
        idx_ref[0, :, h0 : h0 + _BH, :] = center.astype(jnp.int32)

        cb = center.astype(jnp.bfloat16)
        eq = [cb == jnp.bfloat16(a) for a in range(8)]

        acc = None
        for b in range(8):
            if b == 7:
                mb = jnp.where(idf >= 7.0, xc, 0.0)
            else:
                mb = jnp.where(idf == float(b), xc, 0.0)
            mbp = jnp.concatenate([zc, mb, zc], axis=0)
            mv = [
                jnp.concatenate([zw, mbp[:, :, : _W - 1]], axis=2).astype(jnp.bfloat16),
                mbp.astype(jnp.bfloat16),
                jnp.concatenate([mbp[:, :, 1:], zw], axis=2).astype(jnp.bfloat16),
            ]
            rows = []
            for b3 in range(3):
                ra = None
                for a3 in range(3):
                    for d3 in range(3):
                        term = fsb[(a3 * 3 + b3) * 3 + d3] * mv[d3][a3 : a3 + _C, :, :]
                        ra = term if ra is None else ra + term
                rows.append(ra)
            sb = (
                rows[0][:, 0:_BH, :]
                + rows[1][:, 1 : _BH + 1, :]
                + rows[2][:, 2 : _BH + 2, :]
            )
            gb = jnp.where(eq[7], cob[7][b], jnp.bfloat16(0))
            for a in range(6, -1, -1):
                gb = jnp.where(eq[a], cob[a][b], gb)
            term = gb * sb
            acc = term if acc is None else acc + term
        out_ref[0, :, h0 : h0 + _BH, :] = acc.astype(jnp.float32)


def kernel(input, co_matrix, spatial_filter):
    x = input
    minv, maxv = pl.pallas_call(
        _minmax_kernel,
        grid=(2, _N // 2),
        in_specs=[
            pl.BlockSpec((1, _C, _H, _W), lambda i, j: (i * (_N // 2) + j, 0, 0, 0)),
        ],
        out_specs=[
            pl.BlockSpec((1, 1, 128), lambda i, j: (i * (_N // 2) + j, 0, 0)),
            pl.BlockSpec((1, 1, 128), lambda i, j: (i * (_N // 2) + j, 0, 0)),
        ],
        out_shape=[
            jax.ShapeDtypeStruct((_N, 1, 128), jnp.float32),
            jax.ShapeDtypeStruct((_N, 1, 128), jnp.float32),
        ],
        compiler_params=pltpu.CompilerParams(
            dimension_semantics=("parallel", "arbitrary"),
        ),
    )(x)

    out, idx = pl.pallas_call(
        _main_kernel,
        grid=(_N,),
        in_specs=[
            pl.BlockSpec(memory_space=pltpu.SMEM),
            pl.BlockSpec(memory_space=pltpu.SMEM),
            pl.BlockSpec(memory_space=pltpu.VMEM),
            pl.BlockSpec(memory_space=pltpu.VMEM),
            pl.BlockSpec((1, _C, _H, _W), lambda n: (n, 0, 0, 0)),
        ],
        out_specs=[
            pl.BlockSpec((1, _C, _H, _W), lambda n: (n, 0, 0, 0)),
            pl.BlockSpec((1, _C, _H, _W), lambda n: (n, 0, 0, 0)),
        ],
        out_shape=[
            jax.ShapeDtypeStruct((_N, _C, _H, _W), jnp.float32),
            jax.ShapeDtypeStruct((_N, _C, _H, _W), jnp.int32),
        ],
        compiler_params=pltpu.CompilerParams(
            dimension_semantics=("parallel",),
            vmem_limit_bytes=56 * 1024 * 1024,
        ),
    )(co_matrix.reshape(64), spatial_filter.reshape(27), minv, maxv, x)

    return (out, co_matrix, spatial_filter, idx)


# speedup vs baseline: 200.3802x; 1.1211x over previous
import jax
import jax.numpy as jnp
from jax.experimental import pallas as pl
from jax.experimental.pallas import tpu as pltpu

_Q = 8
_K = 3
_N, _C, _H, _W = 8, 16, 256, 256
_BH = 256


def _minmax_kernel(x_ref, min_ref, max_ref):
    xb = x_ref[0].reshape(_C * _H, _W)
    mn = jnp.min(xb, axis=0, keepdims=True)
    mn = jnp.min(mn, axis=1, keepdims=True)
    mx = jnp.max(xb, axis=0, keepdims=True)
    mx = jnp.max(mx, axis=1, keepdims=True)
    min_ref[...] = jnp.broadcast_to(mn[None], (1, 1, 128))
    max_ref[...] = jnp.broadcast_to(mx[None], (1, 1, 128))


def _main_kernel(co_ref, f_ref, minv_ref, maxv_ref, x_ref, out_ref, idx_ref):
    xmin = jnp.min(minv_ref[...])
    xmax = jnp.max(maxv_ref[...])
    cob = [[co_ref[8 * a + b].astype(jnp.bfloat16) for b in range(8)]
           for a in range(8)]
    fsb = [f_ref[k].astype(jnp.bfloat16) for k in range(27)]

    zcb = jnp.zeros((1, _BH + 2, _W), jnp.bfloat16)
    zwb = jnp.zeros((_C + 2, _BH + 2, 1), jnp.bfloat16)
    zrow = jnp.zeros((_C, 1, _W), jnp.float32)

    nchunks = _H // _BH
    for ci in range(nchunks):
        h0 = ci * _BH
        if nchunks == 1:
            xc = jnp.concatenate([zrow, x_ref[0, :, :, :], zrow], axis=1)
        elif ci == 0:
            xc = jnp.concatenate([zrow, x_ref[0, :, : _BH + 1, :]], axis=1)
        elif ci == nchunks - 1:
            xc = jnp.concatenate([x_ref[0, :, h0 - 1 :, :], zrow], axis=1)
        else:
            xc = x_ref[0, :, h0 - 1 : h0 + _BH + 1, :]

        t = (xc - xmin) / xmax * 8.0
        idf = jnp.floor(t)
        center = idf[:, 1 : _BH + 1, :]
        idx_ref[0, :, h0 : h0 + _BH, :] = center.astype(jnp.int32)

        idb = idf.astype(jnp.bfloat16)
        xcb = xc.astype(jnp.bfloat16)
        cb = idb[:, 1 : _BH + 1, :]
        eq = [cb == jnp.bfloat16(a) for a in range(8)]
        zb = jnp.bfloat16(0)

        acc = None
        for b in range(8):
            if b == 7:
                mb = jnp.where(idb >= jnp.bfloat16(7), xcb, zb)
            else:
                mb = jnp.where(idb == jnp.bfloat16(b), xcb, zb)
            mbp = jnp.concatenate([zcb, mb, zcb], axis=0)
            mv = [
                jnp.concatenate([zwb, mbp[:, :, : _W - 1]], axis=2),
                mbp,
                jnp.concatenate([mbp[:, :, 1:], zwb], axis=2),
            ]
            rows = []
            for b3 in range(3):
                ra = None
                for a3 in range(3):
                    for d3 in range(3):
                        term = fsb[(a3 * 3 + b3) * 3 + d3] * mv[d3][a3 : a3 + _C, :, :]
                        ra = term if ra is None else ra + term
                rows.append(ra)
            sb = (
                rows[0][:, 0:_BH, :]
                + rows[1][:, 1 : _BH + 1, :]
                + rows[2][:, 2 : _BH + 2, :]
            )
            gb = jnp.where(eq[7], cob[7][b], jnp.bfloat16(0))
            for a in range(6, -1, -1):
                gb = jnp.where(eq[a], cob[a][b], gb)
            term = gb * sb
            acc = term if acc is None else acc + term
        out_ref[0, :, h0 : h0 + _BH, :] = acc.astype(jnp.float32)


def kernel(input, co_matrix, spatial_filter):
    x = input
    minv, maxv = pl.pallas_call(
        _minmax_kernel,
        grid=(2, _N // 2),
        in_specs=[
            pl.BlockSpec((1, _C, _H, _W), lambda i, j: (i * (_N // 2) + j, 0, 0, 0)),
        ],
        out_specs=[
            pl.BlockSpec((1, 1, 128), lambda i, j: (i * (_N // 2) + j, 0, 0)),
            pl.BlockSpec((1, 1, 128), lambda i, j: (i * (_N // 2) + j, 0, 0)),
        ],
        out_shape=[
            jax.ShapeDtypeStruct((_N, 1, 128), jnp.float32),
            jax.ShapeDtypeStruct((_N, 1, 128), jnp.float32),
        ],
        compiler_params=pltpu.CompilerParams(
            dimension_semantics=("parallel", "arbitrary"),
        ),
    )(x)

    out, idx = pl.pallas_call(
        _main_kernel,
        grid=(_N,),
        in_specs=[
            pl.BlockSpec(memory_space=pltpu.SMEM),
            pl.BlockSpec(memory_space=pltpu.SMEM),
            pl.BlockSpec(memory_space=pltpu.VMEM),
            pl.BlockSpec(memory_space=pltpu.VMEM),
            pl.BlockSpec((1, _C, _H, _W), lambda n: (n, 0, 0, 0)),
        ],
        out_specs=[
            pl.BlockSpec((1, _C, _H, _W), lambda n: (n, 0, 0, 0)),
            pl.BlockSpec((1, _C, _H, _W), lambda n: (n, 0, 0, 0)),
        ],
        out_shape=[
            jax.ShapeDtypeStruct((_N, _C, _H, _W), jnp.float32),
            jax.ShapeDtypeStruct((_N, _C, _H, _W), jnp.int32),
        ],
        compiler_params=pltpu.CompilerParams(
            dimension_semantics=("parallel",),
            vmem_limit_bytes=56 * 1024 * 1024,
        ),
    )(co_matrix.reshape(64), spatial_filter.reshape(27), minv, maxv, x)

    return (out, co_matrix, spatial_filter, idx)
